# baseline (speedup 1.0000x reference)
import functools

import jax
import jax.numpy as jnp
from jax import lax
from jax.experimental import pallas as pl
from jax.experimental.pallas import tpu as pltpu
from jax.experimental.pallas import tpu_sc as plsc

N = 100000
F = 9
V = 100
H = 128
L = 16
C = 16
NW = 32
NCHUNK = N // C


def _sc_body(x_hbm, emb_hbm, out_hbm, tables, raws, outs,
             idx_sem, tab_sem, osems):
    cid = lax.axis_index("c")
    sid = lax.axis_index("s")
    wid = sid * 2 + cid
    nj = (NCHUNK - wid + NW - 1) // NW

    lanes = lax.broadcasted_iota(jnp.int32, (L,), 0)
    cols = [jnp.int32(cc * L) + lanes for cc in range(H // L)]

    def fire_idx(j, b):
        chunk = wid + j * NW
        return pltpu.async_copy(x_hbm.at[pl.ds(chunk * (C * F), C * F)],
                                raws[b], idx_sem)

    def wait_out(b):
        pltpu.make_async_copy(outs[b], out_hbm.at[pl.ds(0, C)],
                              osems[b]).wait()

    @pl.when(nj > 0)
    def _():
        fire_idx(0, 0)
    tcps = [pltpu.async_copy(emb_hbm.at[i], tables[i], tab_sem)
            for i in range(F)]
    for cp in tcps:
        cp.wait()

    def do_chunk(j, b):
        chunk = wid + j * NW

        pltpu.make_async_copy(x_hbm.at[pl.ds(0, C * F)], raws[b],
                              idx_sem).wait()

        @pl.when(j + 1 < nj)
        def _():
            fire_idx(j + 1, 1 - b)

        @pl.when(j >= 2)
        def _():
            wait_out(b)

        def row_step(r, carry):
            rbase = r * F
            idxs = [
                plsc.load_gather(raws[b], [jnp.full((L,), rbase + i,
                                                    jnp.int32)])
                for i in range(F)
            ]
            for cc in range(H // L):
                acc = plsc.load_gather(tables[0], [idxs[0], cols[cc]])
                for i in range(1, F):
                    acc = acc + plsc.load_gather(tables[i],
                                                 [idxs[i], cols[cc]])
                outs[b][r, pl.ds(cc * L, L)] = acc
            return carry

        lax.fori_loop(0, C, row_step, 0)

        pltpu.async_copy(outs[b], out_hbm.at[pl.ds(chunk * C, C)], osems[b])

    def pair_step(jj, carry):
        j0 = jj * 2

        @pl.when(j0 < nj)
        def _():
            do_chunk(j0, 0)

        @pl.when(j0 + 1 < nj)
        def _():
            do_chunk(j0 + 1, 1)

        return carry

    lax.fori_loop(0, (nj + 1) // 2, pair_step, 0)

    for b in range(2):
        @pl.when(nj >= 2 - b)
        def _():
            wait_out(b)


@functools.lru_cache(maxsize=1)
def _build_encoder():
    @functools.partial(
        pl.kernel,
        out_type=jax.ShapeDtypeStruct((N, H), jnp.float32),
        mesh=plsc.VectorSubcoreMesh(core_axis_name="c", subcore_axis_name="s"),
        compiler_params=pltpu.CompilerParams(needs_layout_passes=False),
        scratch_types=(
            [pltpu.VMEM((V, H), jnp.float32) for _ in range(F)]
            + [pltpu.VMEM((C * F,), jnp.int32) for _ in range(2)]
            + [pltpu.VMEM((C, H), jnp.float32) for _ in range(2)]
            + [pltpu.SemaphoreType.DMA, pltpu.SemaphoreType.DMA,
               pltpu.SemaphoreType.DMA, pltpu.SemaphoreType.DMA]
        ),
    )
    def _sc_encoder(x_hbm, emb_hbm, out_hbm, t0, t1, t2, t3, t4, t5, t6, t7,
                    t8, raw0, raw1, o0, o1, idx_sem, tab_sem, osem0, osem1):
        _sc_body(x_hbm, emb_hbm, out_hbm,
                 (t0, t1, t2, t3, t4, t5, t6, t7, t8),
                 (raw0, raw1), (o0, o1), idx_sem, tab_sem, (osem0, osem1))

    return _sc_encoder


def kernel(x, emb):
    return _build_encoder()(x.astype(jnp.int32).reshape(N * F), emb)


# speedup vs baseline: 1.0964x; 1.0964x over previous
import functools

import jax
import jax.numpy as jnp
from jax import lax
from jax.experimental import pallas as pl
from jax.experimental.pallas import tpu as pltpu
from jax.experimental.pallas import tpu_sc as plsc

N = 100000
F = 9
V = 100
H = 128
L = 16
C = 16
NW = 32
NCHUNK = N // C


def _sc_body(x_hbm, emb_hbm, out_hbm, tables, raws, outs,
             idx_sem, tab_sem, osems):
    cid = lax.axis_index("c")
    sid = lax.axis_index("s")
    wid = sid * 2 + cid
    nj = (NCHUNK - wid + NW - 1) // NW

    lanes = lax.broadcasted_iota(jnp.int32, (L,), 0)
    cols = [jnp.int32(cc * L) + lanes for cc in range(H // L)]

    def fire_idx(j, b):
        chunk = wid + j * NW
        return pltpu.async_copy(x_hbm.at[pl.ds(chunk * C, C)],
                                raws[b], idx_sem)

    def wait_out(b):
        pltpu.make_async_copy(outs[b], out_hbm.at[pl.ds(0, C)],
                              osems[b]).wait()

    @pl.when(nj > 0)
    def _():
        fire_idx(0, 0)
    tcps = [pltpu.async_copy(emb_hbm.at[i], tables[i], tab_sem)
            for i in range(F)]
    for cp in tcps:
        cp.wait()

    def do_chunk(j, b):
        chunk = wid + j * NW

        pltpu.make_async_copy(x_hbm.at[pl.ds(0, C)], raws[b],
                              idx_sem).wait()

        @pl.when(j + 1 < nj)
        def _():
            fire_idx(j + 1, 1 - b)

        @pl.when(j >= 2)
        def _():
            wait_out(b)

        def row_step(r, carry):
            idxs = [
                plsc.load_gather(raws[b], [jnp.full((L,), r, jnp.int32),
                                           jnp.full((L,), i, jnp.int32)])
                for i in range(F)
            ]
            for cc in range(H // L):
                acc = plsc.load_gather(tables[0], [idxs[0], cols[cc]])
                for i in range(1, F):
                    acc = acc + plsc.load_gather(tables[i],
                                                 [idxs[i], cols[cc]])
                outs[b][r, pl.ds(cc * L, L)] = acc
            return carry

        lax.fori_loop(0, C, row_step, 0)

        pltpu.async_copy(outs[b], out_hbm.at[pl.ds(chunk * C, C)], osems[b])

    def pair_step(jj, carry):
        j0 = jj * 2

        @pl.when(j0 < nj)
        def _():
            do_chunk(j0, 0)

        @pl.when(j0 + 1 < nj)
        def _():
            do_chunk(j0 + 1, 1)

        return carry

    lax.fori_loop(0, (nj + 1) // 2, pair_step, 0)

    for b in range(2):
        @pl.when(nj >= 2 - b)
        def _():
            wait_out(b)


@functools.lru_cache(maxsize=1)
def _build_encoder():
    @functools.partial(
        pl.kernel,
        out_type=jax.ShapeDtypeStruct((N, H), jnp.float32),
        mesh=plsc.VectorSubcoreMesh(core_axis_name="c", subcore_axis_name="s"),
        compiler_params=pltpu.CompilerParams(needs_layout_passes=False),
        scratch_types=(
            [pltpu.VMEM((V, H), jnp.float32) for _ in range(F)]
            + [pltpu.VMEM((C, F), jnp.int32) for _ in range(2)]
            + [pltpu.VMEM((C, H), jnp.float32) for _ in range(2)]
            + [pltpu.SemaphoreType.DMA, pltpu.SemaphoreType.DMA,
               pltpu.SemaphoreType.DMA, pltpu.SemaphoreType.DMA]
        ),
    )
    def _sc_encoder(x_hbm, emb_hbm, out_hbm, t0, t1, t2, t3, t4, t5, t6, t7,
                    t8, raw0, raw1, o0, o1, idx_sem, tab_sem, osem0, osem1):
        _sc_body(x_hbm, emb_hbm, out_hbm,
                 (t0, t1, t2, t3, t4, t5, t6, t7, t8),
                 (raw0, raw1), (o0, o1), idx_sem, tab_sem, (osem0, osem1))

    return _sc_encoder


def kernel(x, emb):
    return _build_encoder()(x.astype(jnp.int32), emb)
